# baseline (speedup 1.0000x reference)
import functools

import jax
import jax.numpy as jnp
from jax import lax
from jax.experimental import pallas as pl
from jax.experimental.pallas import tpu as pltpu
from jax.experimental.pallas import tpu_sc as plsc

_B = 16384
_F = 26
_V = 100000
_D = 50
_DP = 64
_NUMF = 13
_NUMD = 130
_H = 715

_N = _B * _F
_NC = 2
_NS = 16
_NW = _NC * _NS
_PER_W = _N // _NW
_LPC = 64
_CH = _PER_W // _LPC
_TW = _F * _V * _D // 128
_CPW = _LPC * _DP
_CPR = _CPW // 128
_OUTR = _N * _DP // 128


def _compact_chunk(buf, ph_v, cp, jj, iota):
    @pl.loop(0, _LPC // 16)
    def _(g):
        ph16 = ph_v[jj, pl.ds(g * 16, 16)]
        for c in range(16):
            p = ph16[c]
            r = lax.bitwise_and(p, 15)
            align = pl.multiple_of(lax.bitwise_and(p, 112), 16)
            i = g * 16 + c
            rot = lax.bitwise_and(iota + r, 15)
            keep = iota < 16 - r
            vs = [buf[i, pl.ds(align + 16 * k, 16)] for k in range(5)]
            for k in range(4):
                out_v = jnp.where(keep, vs[k][rot], vs[k + 1][rot])
                w = c * _DP + k * 16
                cp[g * 8 + w // 128, pl.ds(w % 128, 16)] = out_v


def _gather_body(table_hbm, rows_hbm, ph_hbm, out_hbm,
                 rows_v, ph_v, bufs, cps, gsemA, gsemB, osem):
    c = lax.axis_index("c")
    s = lax.axis_index("s")
    wid = s * _NC + c
    wbase = wid * (_CH * _CPR)
    iota = lax.iota(jnp.int32, 16)

    pltpu.sync_copy(rows_hbm.at[wid], rows_v)
    pltpu.sync_copy(ph_hbm.at[wid], ph_v)

    def gather(jj, p):
        pltpu.async_copy(table_hbm.at[rows_v.at[jj, pl.ds(0, _LPC)]],
                         bufs[p].at[:, pl.ds(0, 128)], gsemA[p])
        pltpu.async_copy(table_hbm.at[rows_v.at[jj, pl.ds(_LPC, _LPC)]],
                         bufs[p].at[:, pl.ds(128, 128)], gsemB[p])

    def gather_wait(jj, p):
        pltpu.make_async_copy(table_hbm.at[rows_v.at[jj, pl.ds(0, _LPC)]],
                              bufs[p].at[:, pl.ds(0, 128)], gsemA[p]).wait()
        pltpu.make_async_copy(table_hbm.at[rows_v.at[jj, pl.ds(_LPC, _LPC)]],
                              bufs[p].at[:, pl.ds(128, 128)], gsemB[p]).wait()

    def out_desc(jj, p):
        dst = pl.multiple_of(wbase + jj * _CPR, 8)
        return pltpu.make_async_copy(cps[p], out_hbm.at[pl.ds(dst, _CPR)],
                                     osem[p])

    gather(0, 0)

    @pl.loop(0, _CH, step=2)
    def _(j):
        for ci in range(2):
            jj = j + ci
            p = ci
            gather_wait(jj, p)

            @pl.when(jj + 1 < _CH)
            def _():
                gather(jj + 1, 1 - p)

            @pl.when(jj >= 2)
            def _():
                out_desc(jj - 2, p).wait()

            _compact_chunk(bufs[p], ph_v, cps[p], jj, iota)
            out_desc(jj, p).start()

    out_desc(_CH - 2, 0).wait()
    out_desc(_CH - 1, 1).wait()


_sc_gather = functools.partial(
    pl.kernel,
    out_type=jax.ShapeDtypeStruct((_OUTR, 128), jnp.float32),
    mesh=plsc.VectorSubcoreMesh(core_axis_name="c", subcore_axis_name="s"),
    scratch_types=[
        pltpu.VMEM((_CH, 128), jnp.int32),
        pltpu.VMEM((_CH, 128), jnp.int32),
        [pltpu.VMEM((_LPC, 256), jnp.float32),
         pltpu.VMEM((_LPC, 256), jnp.float32)],
        [pltpu.VMEM((_CPR, 128), jnp.float32),
         pltpu.VMEM((_CPR, 128), jnp.float32)],
        [pltpu.SemaphoreType.DMA, pltpu.SemaphoreType.DMA],
        [pltpu.SemaphoreType.DMA, pltpu.SemaphoreType.DMA],
        [pltpu.SemaphoreType.DMA, pltpu.SemaphoreType.DMA],
    ],
)(_gather_body)


_BM = 1024
_KC = _F * _DP


def _mlp_body(xc_ref, xn_ref, wnum_ref, bnum_ref, w1c_ref, w1n_ref, b1_ref,
              w2_ref, b2_ref, out_ref):
    x_n = lax.dot_general(xn_ref[...], wnum_ref[...],
                          (((1,), (1,)), ((), ()))) + bnum_ref[...]
    hc = lax.dot_general(xc_ref[...], w1c_ref[...], (((1,), (1,)), ((), ())))
    hn = lax.dot_general(x_n, w1n_ref[...], (((1,), (1,)), ((), ())))
    h = jnp.maximum(hc + hn + b1_ref[...], 0.0)
    out_ref[...] = lax.dot_general(h, w2_ref[...],
                                   (((1,), (1,)), ((), ()))) + b2_ref[...]


def _mlp(x_c, x_num, W_num, b_num, W1c, W1n, b1, W2, b2):
    return pl.pallas_call(
        _mlp_body,
        grid=(_B // _BM,),
        in_specs=[
            pl.BlockSpec((_BM, _KC), lambda i: (i, 0)),
            pl.BlockSpec((_BM, _NUMF), lambda i: (i, 0)),
            pl.BlockSpec((_NUMD, _NUMF), lambda i: (0, 0)),
            pl.BlockSpec((1, _NUMD), lambda i: (0, 0)),
            pl.BlockSpec((_H, _KC), lambda i: (0, 0)),
            pl.BlockSpec((_H, _NUMD), lambda i: (0, 0)),
            pl.BlockSpec((1, _H), lambda i: (0, 0)),
            pl.BlockSpec((3, _H), lambda i: (0, 0)),
            pl.BlockSpec((1, 3), lambda i: (0, 0)),
        ],
        out_specs=pl.BlockSpec((_BM, 3), lambda i: (i, 0)),
        out_shape=jax.ShapeDtypeStruct((_B, 3), jnp.float32),
        compiler_params=pltpu.CompilerParams(
            dimension_semantics=("arbitrary",),
        ),
    )(x_c, x_num, W_num, b_num, W1c, W1n, b1, W2, b2)


def kernel(x_cat, x_num, tables, W_num, b_num, W1, b1, W2, b2):
    table128 = tables.reshape(_TW, 128)
    g = (x_cat.astype(jnp.int32)
         + (jnp.arange(_F, dtype=jnp.int32) * _V)[None, :]).reshape(-1)
    w50 = g * _D
    r0 = w50 >> 7
    r1 = jnp.minimum(r0 + 1, _TW - 1)
    phase = w50 & 127
    rows = jnp.concatenate([r0.reshape(_NW, _CH, _LPC),
                            r1.reshape(_NW, _CH, _LPC)], axis=-1)
    ph = jnp.concatenate([phase.reshape(_NW, _CH, _LPC),
                          jnp.zeros((_NW, _CH, 128 - _LPC), jnp.int32)],
                         axis=-1)
    flat = _sc_gather(table128, rows, ph)
    x_c = flat.reshape(_B, _KC)
    W1c = jnp.pad(W1[:, :_F * _D].reshape(_H, _F, _D),
                  ((0, 0), (0, 0), (0, _DP - _D))).reshape(_H, _KC)
    W1n = W1[:, _F * _D:]
    return _mlp(x_c, x_num, W_num, b_num.reshape(1, _NUMD), W1c, W1n,
                b1.reshape(1, _H), W2, b2.reshape(1, 3))


# speedup vs baseline: 1.0217x; 1.0217x over previous
import functools

import jax
import jax.numpy as jnp
from jax import lax
from jax.experimental import pallas as pl
from jax.experimental.pallas import tpu as pltpu
from jax.experimental.pallas import tpu_sc as plsc

_B = 16384
_F = 26
_V = 100000
_D = 50
_DP = 64
_NUMF = 13
_NUMD = 130
_H = 715

_N = _B * _F
_NC = 2
_NS = 16
_NW = _NC * _NS
_PER_W = _N // _NW
_LPC = 128
_CH = _PER_W // _LPC
_TW = _F * _V * _D // 128
_CPW = _LPC * _DP
_CPR = _CPW // 128
_OUTR = _N * _DP // 128


def _compact_chunk(buf, ph_v, cp, jj, iota):
    @pl.loop(0, _LPC // 16)
    def _(g):
        ph16 = ph_v[jj, pl.ds(g * 16, 16)]
        for c in range(16):
            p = ph16[c]
            r = lax.bitwise_and(p, 15)
            align = pl.multiple_of(lax.bitwise_and(p, 112), 16)
            i = g * 16 + c
            rot = lax.bitwise_and(iota + r, 15)
            keep = iota < 16 - r
            vs = [buf[i, pl.ds(align + 16 * k, 16)] for k in range(5)]
            for k in range(4):
                out_v = jnp.where(keep, vs[k][rot], vs[k + 1][rot])
                w = c * _DP + k * 16
                cp[g * 8 + w // 128, pl.ds(w % 128, 16)] = out_v


def _gather_body(table_hbm, rows_hbm, ph_hbm, out_hbm,
                 rows_v, ph_v, bufs, cps, gsemA, gsemB, osem):
    c = lax.axis_index("c")
    s = lax.axis_index("s")
    wid = s * _NC + c
    wbase = wid * (_CH * _CPR)
    iota = lax.iota(jnp.int32, 16)

    pltpu.sync_copy(rows_hbm.at[wid], rows_v)
    pltpu.sync_copy(ph_hbm.at[wid], ph_v)

    def gather(jj, p):
        pltpu.async_copy(table_hbm.at[rows_v.at[jj, pl.ds(0, _LPC)]],
                         bufs[p].at[:, pl.ds(0, 128)], gsemA[p])
        pltpu.async_copy(table_hbm.at[rows_v.at[jj, pl.ds(_LPC, _LPC)]],
                         bufs[p].at[:, pl.ds(128, 128)], gsemB[p])

    def gather_wait(jj, p):
        pltpu.make_async_copy(table_hbm.at[rows_v.at[jj, pl.ds(0, _LPC)]],
                              bufs[p].at[:, pl.ds(0, 128)], gsemA[p]).wait()
        pltpu.make_async_copy(table_hbm.at[rows_v.at[jj, pl.ds(_LPC, _LPC)]],
                              bufs[p].at[:, pl.ds(128, 128)], gsemB[p]).wait()

    def out_desc(jj, p):
        dst = pl.multiple_of(wbase + jj * _CPR, 8)
        return pltpu.make_async_copy(cps[p], out_hbm.at[pl.ds(dst, _CPR)],
                                     osem[p])

    gather(0, 0)

    @pl.loop(0, _CH, step=2)
    def _(j):
        for ci in range(2):
            jj = j + ci
            p = ci
            gather_wait(jj, p)

            @pl.when(jj + 1 < _CH)
            def _():
                gather(jj + 1, 1 - p)

            @pl.when(jj >= 2)
            def _():
                out_desc(jj - 2, p).wait()

            _compact_chunk(bufs[p], ph_v, cps[p], jj, iota)
            out_desc(jj, p).start()

    out_desc(_CH - 2, 0).wait()
    out_desc(_CH - 1, 1).wait()


_sc_gather = functools.partial(
    pl.kernel,
    out_type=jax.ShapeDtypeStruct((_OUTR, 128), jnp.float32),
    mesh=plsc.VectorSubcoreMesh(core_axis_name="c", subcore_axis_name="s"),
    scratch_types=[
        pltpu.VMEM((_CH, 2 * _LPC), jnp.int32),
        pltpu.VMEM((_CH, _LPC), jnp.int32),
        [pltpu.VMEM((_LPC, 256), jnp.float32),
         pltpu.VMEM((_LPC, 256), jnp.float32)],
        [pltpu.VMEM((_CPR, 128), jnp.float32),
         pltpu.VMEM((_CPR, 128), jnp.float32)],
        [pltpu.SemaphoreType.DMA, pltpu.SemaphoreType.DMA],
        [pltpu.SemaphoreType.DMA, pltpu.SemaphoreType.DMA],
        [pltpu.SemaphoreType.DMA, pltpu.SemaphoreType.DMA],
    ],
)(_gather_body)


_BM = 1024
_KC = _F * _DP


def _mlp_body(xc_ref, xn_ref, wnum_ref, bnum_ref, w1c_ref, w1n_ref, b1_ref,
              w2_ref, b2_ref, out_ref):
    x_n = lax.dot_general(xn_ref[...], wnum_ref[...],
                          (((1,), (1,)), ((), ()))) + bnum_ref[...]
    hc = lax.dot_general(xc_ref[...], w1c_ref[...], (((1,), (1,)), ((), ())))
    hn = lax.dot_general(x_n, w1n_ref[...], (((1,), (1,)), ((), ())))
    h = jnp.maximum(hc + hn + b1_ref[...], 0.0)
    out_ref[...] = lax.dot_general(h, w2_ref[...],
                                   (((1,), (1,)), ((), ()))) + b2_ref[...]


def _mlp(x_c, x_num, W_num, b_num, W1c, W1n, b1, W2, b2):
    return pl.pallas_call(
        _mlp_body,
        grid=(_B // _BM,),
        in_specs=[
            pl.BlockSpec((_BM, _KC), lambda i: (i, 0)),
            pl.BlockSpec((_BM, _NUMF), lambda i: (i, 0)),
            pl.BlockSpec((_NUMD, _NUMF), lambda i: (0, 0)),
            pl.BlockSpec((1, _NUMD), lambda i: (0, 0)),
            pl.BlockSpec((_H, _KC), lambda i: (0, 0)),
            pl.BlockSpec((_H, _NUMD), lambda i: (0, 0)),
            pl.BlockSpec((1, _H), lambda i: (0, 0)),
            pl.BlockSpec((3, _H), lambda i: (0, 0)),
            pl.BlockSpec((1, 3), lambda i: (0, 0)),
        ],
        out_specs=pl.BlockSpec((_BM, 3), lambda i: (i, 0)),
        out_shape=jax.ShapeDtypeStruct((_B, 3), jnp.float32),
        compiler_params=pltpu.CompilerParams(
            dimension_semantics=("arbitrary",),
        ),
    )(x_c, x_num, W_num, b_num, W1c, W1n, b1, W2, b2)


def kernel(x_cat, x_num, tables, W_num, b_num, W1, b1, W2, b2):
    table128 = tables.reshape(_TW, 128)
    g = (x_cat.astype(jnp.int32)
         + (jnp.arange(_F, dtype=jnp.int32) * _V)[None, :]).reshape(-1)
    w50 = g * _D
    r0 = w50 >> 7
    r1 = jnp.minimum(r0 + 1, _TW - 1)
    phase = w50 & 127
    rows = jnp.concatenate([r0.reshape(_NW, _CH, _LPC),
                            r1.reshape(_NW, _CH, _LPC)], axis=-1)
    ph = phase.reshape(_NW, _CH, _LPC)
    flat = _sc_gather(table128, rows, ph)
    x_c = flat.reshape(_B, _KC)
    W1c = jnp.pad(W1[:, :_F * _D].reshape(_H, _F, _D),
                  ((0, 0), (0, 0), (0, _DP - _D))).reshape(_H, _KC)
    W1n = W1[:, _F * _D:]
    return _mlp(x_c, x_num, W_num, b_num.reshape(1, _NUMD), W1c, W1n,
                b1.reshape(1, _H), W2, b2.reshape(1, 3))


# speedup vs baseline: 1.2394x; 1.2131x over previous
import functools

import jax
import jax.numpy as jnp
from jax import lax
from jax.experimental import pallas as pl
from jax.experimental.pallas import tpu as pltpu
from jax.experimental.pallas import tpu_sc as plsc

_B = 16384
_F = 26
_V = 100000
_D = 50
_DP = 64
_NUMF = 13
_NUMD = 130
_H = 715

_N = _B * _F
_NC = 2
_NS = 16
_NW = _NC * _NS
_PER_W = _N // _NW
_LPC = 128
_CH = _PER_W // _LPC
_VC = 1024
_NB = -(-_V // _VC)
_RPB = _VC // 2
_RPF = _NB * _RPB
_TR = _F * _RPF
_CPW = _LPC * _DP
_CPR = _CPW // 128
_OUTR = _N * _DP // 128


def _repack_body(tin_ref, out_ref):
    x = tin_ref[0]
    xt = jnp.swapaxes(x, 0, 1)
    out_ref[:, 0:_D] = xt[0:_RPB]
    out_ref[:, _D:_DP] = jnp.zeros((_RPB, _DP - _D), jnp.float32)
    out_ref[:, _DP:_DP + _D] = xt[_RPB:_VC]
    out_ref[:, _DP + _D:128] = jnp.zeros((_RPB, _DP - _D), jnp.float32)


def _repack(tablesT):
    return pl.pallas_call(
        _repack_body,
        grid=(_F, _NB),
        in_specs=[pl.BlockSpec((1, _D, _VC), lambda f, v: (f, 0, v))],
        out_specs=pl.BlockSpec((_RPB, 128), lambda f, v: (f * _NB + v, 0)),
        out_shape=jax.ShapeDtypeStruct((_TR, 128), jnp.float32),
        compiler_params=pltpu.CompilerParams(
            dimension_semantics=("arbitrary", "arbitrary"),
        ),
    )(tablesT)


def _copy_chunk(buf, off_v, cp, jj, iota):
    @pl.loop(0, _LPC // 16)
    def _(g):
        off16 = off_v[jj, pl.ds(g * 16, 16)]
        for c in range(16):
            off = pl.multiple_of(off16[c], 16)
            i = g * 16 + c
            for k in range(4):
                v = buf[i, pl.ds(off + 16 * k, 16)]
                w = c * _DP + k * 16
                cp[g * 8 + w // 128, pl.ds(w % 128, 16)] = v


def _gather_body(table_hbm, rows_hbm, off_hbm, out_hbm,
                 rows_v, off_v, bufs, cps, gsem, osem):
    c = lax.axis_index("c")
    s = lax.axis_index("s")
    wid = s * _NC + c
    wbase = wid * (_CH * _CPR)
    iota = lax.iota(jnp.int32, 16)

    pltpu.sync_copy(rows_hbm.at[wid], rows_v)
    pltpu.sync_copy(off_hbm.at[wid], off_v)

    def gather(jj, p):
        pltpu.async_copy(table_hbm.at[rows_v.at[jj]], bufs[p], gsem[p])

    def gather_wait(jj, p):
        pltpu.make_async_copy(table_hbm.at[rows_v.at[jj]], bufs[p],
                              gsem[p]).wait()

    def out_desc(jj, p):
        dst = pl.multiple_of(wbase + jj * _CPR, 8)
        return pltpu.make_async_copy(cps[p], out_hbm.at[pl.ds(dst, _CPR)],
                                     osem[p])

    gather(0, 0)

    @pl.loop(0, _CH, step=2)
    def _(j):
        for ci in range(2):
            jj = j + ci
            p = ci
            gather_wait(jj, p)

            @pl.when(jj + 1 < _CH)
            def _():
                gather(jj + 1, 1 - p)

            @pl.when(jj >= 2)
            def _():
                out_desc(jj - 2, p).wait()

            _copy_chunk(bufs[p], off_v, cps[p], jj, iota)
            out_desc(jj, p).start()

    out_desc(_CH - 2, 0).wait()
    out_desc(_CH - 1, 1).wait()


_sc_gather = functools.partial(
    pl.kernel,
    out_type=jax.ShapeDtypeStruct((_OUTR, 128), jnp.float32),
    mesh=plsc.VectorSubcoreMesh(core_axis_name="c", subcore_axis_name="s"),
    scratch_types=[
        pltpu.VMEM((_CH, _LPC), jnp.int32),
        pltpu.VMEM((_CH, _LPC), jnp.int32),
        [pltpu.VMEM((_LPC, 128), jnp.float32),
         pltpu.VMEM((_LPC, 128), jnp.float32)],
        [pltpu.VMEM((_CPR, 128), jnp.float32),
         pltpu.VMEM((_CPR, 128), jnp.float32)],
        [pltpu.SemaphoreType.DMA, pltpu.SemaphoreType.DMA],
        [pltpu.SemaphoreType.DMA, pltpu.SemaphoreType.DMA],
    ],
)(_gather_body)


_BM = 1024
_KC = _F * _DP


def _mlp_body(xc_ref, xn_ref, wnum_ref, bnum_ref, w1c_ref, w1n_ref, b1_ref,
              w2_ref, b2_ref, out_ref):
    x_n = lax.dot_general(xn_ref[...], wnum_ref[...],
                          (((1,), (1,)), ((), ()))) + bnum_ref[...]
    hc = lax.dot_general(xc_ref[...], w1c_ref[...], (((1,), (1,)), ((), ())))
    hn = lax.dot_general(x_n, w1n_ref[...], (((1,), (1,)), ((), ())))
    h = jnp.maximum(hc + hn + b1_ref[...], 0.0)
    out_ref[...] = lax.dot_general(h, w2_ref[...],
                                   (((1,), (1,)), ((), ()))) + b2_ref[...]


def _mlp(x_c, x_num, W_num, b_num, W1c, W1n, b1, W2, b2):
    return pl.pallas_call(
        _mlp_body,
        grid=(_B // _BM,),
        in_specs=[
            pl.BlockSpec((_BM, _KC), lambda i: (i, 0)),
            pl.BlockSpec((_BM, _NUMF), lambda i: (i, 0)),
            pl.BlockSpec((_NUMD, _NUMF), lambda i: (0, 0)),
            pl.BlockSpec((1, _NUMD), lambda i: (0, 0)),
            pl.BlockSpec((_H, _KC), lambda i: (0, 0)),
            pl.BlockSpec((_H, _NUMD), lambda i: (0, 0)),
            pl.BlockSpec((1, _H), lambda i: (0, 0)),
            pl.BlockSpec((3, _H), lambda i: (0, 0)),
            pl.BlockSpec((1, 3), lambda i: (0, 0)),
        ],
        out_specs=pl.BlockSpec((_BM, 3), lambda i: (i, 0)),
        out_shape=jax.ShapeDtypeStruct((_B, 3), jnp.float32),
        compiler_params=pltpu.CompilerParams(
            dimension_semantics=("arbitrary",),
        ),
    )(x_c, x_num, W_num, b_num, W1c, W1n, b1, W2, b2)


def kernel(x_cat, x_num, tables, W_num, b_num, W1, b1, W2, b2):
    tablesT = jnp.swapaxes(tables, 1, 2)
    table128 = _repack(tablesT)
    v = x_cat.astype(jnp.int32)
    f = jnp.arange(_F, dtype=jnp.int32)[None, :]
    row = (f * _RPF + (v // _VC) * _RPB + (v % _RPB)).reshape(-1)
    off = (((v % _VC) // _RPB) * _DP).reshape(-1)
    rows = row.reshape(_NW, _CH, _LPC)
    offs = off.reshape(_NW, _CH, _LPC)
    flat = _sc_gather(table128, rows, offs)
    x_c = flat.reshape(_B, _KC)
    W1c = jnp.pad(W1[:, :_F * _D].reshape(_H, _F, _D),
                  ((0, 0), (0, 0), (0, _DP - _D))).reshape(_H, _KC)
    W1n = W1[:, _F * _D:]
    return _mlp(x_c, x_num, W_num, b_num.reshape(1, _NUMD), W1c, W1n,
                b1.reshape(1, _H), W2, b2.reshape(1, 3))


# speedup vs baseline: 2.1544x; 1.7383x over previous
import functools

import jax
import jax.numpy as jnp
from jax import lax
from jax.experimental import pallas as pl
from jax.experimental.pallas import tpu as pltpu
from jax.experimental.pallas import tpu_sc as plsc

_B = 16384
_F = 26
_V = 100000
_D = 50
_DP = 64
_NUMF = 13
_NUMD = 130
_H = 715

_N = _B * _F
_NC = 2
_NS = 16
_NW = _NC * _NS
_PER_W = _N // _NW
_LPC = 128
_CH = _PER_W // _LPC
_VC = 4096
_NB = -(-_V // _VC)
_RPB = _VC // 2
_RPF = _NB * _RPB
_TR = _F * _RPF
_CPW = _LPC * _DP
_CPR = _CPW // 128
_OUTR = _N * _DP // 128


def _repack_body(tin_ref, out_ref):
    x = tin_ref[0]
    xt = jnp.swapaxes(x, 0, 1)
    out_ref[:, 0:_D] = xt[0:_RPB]
    out_ref[:, _D:_DP] = jnp.zeros((_RPB, _DP - _D), jnp.float32)
    out_ref[:, _DP:_DP + _D] = xt[_RPB:_VC]
    out_ref[:, _DP + _D:128] = jnp.zeros((_RPB, _DP - _D), jnp.float32)


def _repack(tablesT):
    return pl.pallas_call(
        _repack_body,
        grid=(_F, _NB),
        in_specs=[pl.BlockSpec((1, _D, _VC), lambda f, v: (f, 0, v))],
        out_specs=pl.BlockSpec((_RPB, 128), lambda f, v: (f * _NB + v, 0)),
        out_shape=jax.ShapeDtypeStruct((_TR, 128), jnp.float32),
        compiler_params=pltpu.CompilerParams(
            dimension_semantics=("arbitrary", "arbitrary"),
        ),
    )(tablesT)


def _copy_chunk(buf, off_v, cp, jj, iota):
    @pl.loop(0, _LPC // 16)
    def _(g):
        off16 = off_v[jj, pl.ds(g * 16, 16)]
        for c in range(16):
            off = pl.multiple_of(off16[c], 16)
            i = g * 16 + c
            for k in range(4):
                v = buf[i, pl.ds(off + 16 * k, 16)]
                w = c * _DP + k * 16
                cp[g * 8 + w // 128, pl.ds(w % 128, 16)] = v


def _gather_body(table_hbm, rows_hbm, off_hbm, out_hbm,
                 rows_v, off_v, bufs, cps, gsem, osem):
    c = lax.axis_index("c")
    s = lax.axis_index("s")
    wid = s * _NC + c
    wbase = wid * (_CH * _CPR)
    iota = lax.iota(jnp.int32, 16)

    pltpu.sync_copy(rows_hbm.at[wid], rows_v)
    pltpu.sync_copy(off_hbm.at[wid], off_v)

    def gather(jj, p):
        pltpu.async_copy(table_hbm.at[rows_v.at[jj]], bufs[p], gsem[p])

    def gather_wait(jj, p):
        pltpu.make_async_copy(table_hbm.at[rows_v.at[jj]], bufs[p],
                              gsem[p]).wait()

    def out_desc(jj, p):
        dst = pl.multiple_of(wbase + jj * _CPR, 8)
        return pltpu.make_async_copy(cps[p], out_hbm.at[pl.ds(dst, _CPR)],
                                     osem[p])

    gather(0, 0)

    @pl.loop(0, _CH, step=2)
    def _(j):
        for ci in range(2):
            jj = j + ci
            p = ci
            gather_wait(jj, p)

            @pl.when(jj + 1 < _CH)
            def _():
                gather(jj + 1, 1 - p)

            @pl.when(jj >= 2)
            def _():
                out_desc(jj - 2, p).wait()

            _copy_chunk(bufs[p], off_v, cps[p], jj, iota)
            out_desc(jj, p).start()

    out_desc(_CH - 2, 0).wait()
    out_desc(_CH - 1, 1).wait()


_sc_gather = functools.partial(
    pl.kernel,
    out_type=jax.ShapeDtypeStruct((_OUTR, 128), jnp.float32),
    mesh=plsc.VectorSubcoreMesh(core_axis_name="c", subcore_axis_name="s"),
    scratch_types=[
        pltpu.VMEM((_CH, _LPC), jnp.int32),
        pltpu.VMEM((_CH, _LPC), jnp.int32),
        [pltpu.VMEM((_LPC, 128), jnp.float32),
         pltpu.VMEM((_LPC, 128), jnp.float32)],
        [pltpu.VMEM((_CPR, 128), jnp.float32),
         pltpu.VMEM((_CPR, 128), jnp.float32)],
        [pltpu.SemaphoreType.DMA, pltpu.SemaphoreType.DMA],
        [pltpu.SemaphoreType.DMA, pltpu.SemaphoreType.DMA],
    ],
)(_gather_body)


_BM = 1024
_KC = _F * _DP


def _mlp_body(xc_ref, xn_ref, wnum_ref, bnum_ref, w1c_ref, w1n_ref, b1_ref,
              w2_ref, b2_ref, out_ref):
    x_n = lax.dot_general(xn_ref[...], wnum_ref[...],
                          (((1,), (1,)), ((), ()))) + bnum_ref[...]
    hc = lax.dot_general(xc_ref[...], w1c_ref[...], (((1,), (1,)), ((), ())))
    hn = lax.dot_general(x_n, w1n_ref[...], (((1,), (1,)), ((), ())))
    h = jnp.maximum(hc + hn + b1_ref[...], 0.0)
    out_ref[...] = lax.dot_general(h, w2_ref[...],
                                   (((1,), (1,)), ((), ()))) + b2_ref[...]


def _mlp(x_c, x_num, W_num, b_num, W1c, W1n, b1, W2, b2):
    return pl.pallas_call(
        _mlp_body,
        grid=(_B // _BM,),
        in_specs=[
            pl.BlockSpec((_BM, _KC), lambda i: (i, 0)),
            pl.BlockSpec((_BM, _NUMF), lambda i: (i, 0)),
            pl.BlockSpec((_NUMD, _NUMF), lambda i: (0, 0)),
            pl.BlockSpec((1, _NUMD), lambda i: (0, 0)),
            pl.BlockSpec((_H, _KC), lambda i: (0, 0)),
            pl.BlockSpec((_H, _NUMD), lambda i: (0, 0)),
            pl.BlockSpec((1, _H), lambda i: (0, 0)),
            pl.BlockSpec((3, _H), lambda i: (0, 0)),
            pl.BlockSpec((1, 3), lambda i: (0, 0)),
        ],
        out_specs=pl.BlockSpec((_BM, 3), lambda i: (i, 0)),
        out_shape=jax.ShapeDtypeStruct((_B, 3), jnp.float32),
        compiler_params=pltpu.CompilerParams(
            dimension_semantics=("arbitrary",),
        ),
    )(x_c, x_num, W_num, b_num, W1c, W1n, b1, W2, b2)


def kernel(x_cat, x_num, tables, W_num, b_num, W1, b1, W2, b2):
    tablesT = jnp.swapaxes(tables, 1, 2)
    table128 = _repack(tablesT)
    v = x_cat.astype(jnp.int32)
    f = jnp.arange(_F, dtype=jnp.int32)[None, :]
    row = (f * _RPF + (v // _VC) * _RPB + (v % _RPB)).reshape(-1)
    off = (((v % _VC) // _RPB) * _DP).reshape(-1)
    rows = row.reshape(_NW, _CH, _LPC)
    offs = off.reshape(_NW, _CH, _LPC)
    flat = _sc_gather(table128, rows, offs)
    x_c = flat.reshape(_B, _KC)
    W1c = jnp.pad(W1[:, :_F * _D].reshape(_H, _F, _D),
                  ((0, 0), (0, 0), (0, _DP - _D))).reshape(_H, _KC)
    W1n = W1[:, _F * _D:]
    return _mlp(x_c, x_num, W_num, b_num.reshape(1, _NUMD), W1c, W1n,
                b1.reshape(1, _H), W2, b2.reshape(1, 3))


# speedup vs baseline: 2.4589x; 1.1413x over previous
import functools

import jax
import jax.numpy as jnp
from jax import lax
from jax.experimental import pallas as pl
from jax.experimental.pallas import tpu as pltpu
from jax.experimental.pallas import tpu_sc as plsc

_B = 16384
_F = 26
_V = 100000
_D = 50
_DP = 64
_NUMF = 13
_NUMD = 130
_H = 715

_N = _B * _F
_NC = 2
_NS = 16
_NW = _NC * _NS
_PER_W = _N // _NW
_LPC = 128
_CH = _PER_W // _LPC
_VC = 8192
_NB = -(-_V // _VC)
_RPB = _VC // 2
_RPF = _NB * _RPB
_TR = _F * _RPF
_CPW = _LPC * _DP
_CPR = _CPW // 128
_OUTR = _N * _DP // 128


def _repack_body(tin_ref, out_ref):
    x = tin_ref[0]
    xt = jnp.swapaxes(x, 0, 1)
    out_ref[:, 0:_D] = xt[0:_RPB]
    out_ref[:, _D:_DP] = jnp.zeros((_RPB, _DP - _D), jnp.float32)
    out_ref[:, _DP:_DP + _D] = xt[_RPB:_VC]
    out_ref[:, _DP + _D:128] = jnp.zeros((_RPB, _DP - _D), jnp.float32)


def _repack(tablesT):
    return pl.pallas_call(
        _repack_body,
        grid=(_F, _NB),
        in_specs=[pl.BlockSpec((1, _D, _VC), lambda f, v: (f, 0, v))],
        out_specs=pl.BlockSpec((_RPB, 128), lambda f, v: (f * _NB + v, 0)),
        out_shape=jax.ShapeDtypeStruct((_TR, 128), jnp.float32),
        compiler_params=pltpu.CompilerParams(
            dimension_semantics=("arbitrary", "arbitrary"),
        ),
    )(tablesT)


def _copy_chunk(buf, off_v, cp, jj, iota):
    @pl.loop(0, _LPC // 16)
    def _(g):
        off16 = off_v[jj, pl.ds(g * 16, 16)]
        for c in range(16):
            off = pl.multiple_of(off16[c], 16)
            i = g * 16 + c
            for k in range(4):
                v = buf[i, pl.ds(off + 16 * k, 16)]
                w = c * _DP + k * 16
                cp[g * 8 + w // 128, pl.ds(w % 128, 16)] = v


def _gather_body(table_hbm, rows_hbm, off_hbm, out_hbm,
                 rows_v, off_v, bufs, cps, gsem, osem):
    c = lax.axis_index("c")
    s = lax.axis_index("s")
    wid = s * _NC + c
    wbase = wid * (_CH * _CPR)
    iota = lax.iota(jnp.int32, 16)

    pltpu.sync_copy(rows_hbm.at[wid], rows_v)
    pltpu.sync_copy(off_hbm.at[wid], off_v)

    def gather(jj, p):
        pltpu.async_copy(table_hbm.at[rows_v.at[jj]], bufs[p], gsem[p])

    def gather_wait(jj, p):
        pltpu.make_async_copy(table_hbm.at[rows_v.at[jj]], bufs[p],
                              gsem[p]).wait()

    def out_desc(jj, p):
        dst = pl.multiple_of(wbase + jj * _CPR, 8)
        return pltpu.make_async_copy(cps[p], out_hbm.at[pl.ds(dst, _CPR)],
                                     osem[p])

    gather(0, 0)

    @pl.loop(0, _CH, step=2)
    def _(j):
        for ci in range(2):
            jj = j + ci
            p = ci
            gather_wait(jj, p)

            @pl.when(jj + 1 < _CH)
            def _():
                gather(jj + 1, 1 - p)

            @pl.when(jj >= 2)
            def _():
                out_desc(jj - 2, p).wait()

            _copy_chunk(bufs[p], off_v, cps[p], jj, iota)
            out_desc(jj, p).start()

    out_desc(_CH - 2, 0).wait()
    out_desc(_CH - 1, 1).wait()


_sc_gather = functools.partial(
    pl.kernel,
    out_type=jax.ShapeDtypeStruct((_OUTR, 128), jnp.float32),
    mesh=plsc.VectorSubcoreMesh(core_axis_name="c", subcore_axis_name="s"),
    scratch_types=[
        pltpu.VMEM((_CH, _LPC), jnp.int32),
        pltpu.VMEM((_CH, _LPC), jnp.int32),
        [pltpu.VMEM((_LPC, 128), jnp.float32),
         pltpu.VMEM((_LPC, 128), jnp.float32)],
        [pltpu.VMEM((_CPR, 128), jnp.float32),
         pltpu.VMEM((_CPR, 128), jnp.float32)],
        [pltpu.SemaphoreType.DMA, pltpu.SemaphoreType.DMA],
        [pltpu.SemaphoreType.DMA, pltpu.SemaphoreType.DMA],
    ],
)(_gather_body)


_BM = 1024
_KC = _F * _DP


def _mlp_body(xc_ref, xn_ref, wnum_ref, bnum_ref, w1c_ref, w1n_ref, b1_ref,
              w2_ref, b2_ref, out_ref):
    x_n = lax.dot_general(xn_ref[...], wnum_ref[...],
                          (((1,), (1,)), ((), ()))) + bnum_ref[...]
    hc = lax.dot_general(xc_ref[...], w1c_ref[...], (((1,), (1,)), ((), ())))
    hn = lax.dot_general(x_n, w1n_ref[...], (((1,), (1,)), ((), ())))
    h = jnp.maximum(hc + hn + b1_ref[...], 0.0)
    out_ref[...] = lax.dot_general(h, w2_ref[...],
                                   (((1,), (1,)), ((), ()))) + b2_ref[...]


def _mlp(x_c, x_num, W_num, b_num, W1c, W1n, b1, W2, b2):
    return pl.pallas_call(
        _mlp_body,
        grid=(_B // _BM,),
        in_specs=[
            pl.BlockSpec((_BM, _KC), lambda i: (i, 0)),
            pl.BlockSpec((_BM, _NUMF), lambda i: (i, 0)),
            pl.BlockSpec((_NUMD, _NUMF), lambda i: (0, 0)),
            pl.BlockSpec((1, _NUMD), lambda i: (0, 0)),
            pl.BlockSpec((_H, _KC), lambda i: (0, 0)),
            pl.BlockSpec((_H, _NUMD), lambda i: (0, 0)),
            pl.BlockSpec((1, _H), lambda i: (0, 0)),
            pl.BlockSpec((3, _H), lambda i: (0, 0)),
            pl.BlockSpec((1, 3), lambda i: (0, 0)),
        ],
        out_specs=pl.BlockSpec((_BM, 3), lambda i: (i, 0)),
        out_shape=jax.ShapeDtypeStruct((_B, 3), jnp.float32),
        compiler_params=pltpu.CompilerParams(
            dimension_semantics=("arbitrary",),
        ),
    )(x_c, x_num, W_num, b_num, W1c, W1n, b1, W2, b2)


def kernel(x_cat, x_num, tables, W_num, b_num, W1, b1, W2, b2):
    tablesT = jnp.swapaxes(tables, 1, 2)
    table128 = _repack(tablesT)
    v = x_cat.astype(jnp.int32)
    f = jnp.arange(_F, dtype=jnp.int32)[None, :]
    row = (f * _RPF + (v // _VC) * _RPB + (v % _RPB)).reshape(-1)
    off = (((v % _VC) // _RPB) * _DP).reshape(-1)
    rows = row.reshape(_NW, _CH, _LPC)
    offs = off.reshape(_NW, _CH, _LPC)
    flat = _sc_gather(table128, rows, offs)
    x_c = flat.reshape(_B, _KC)
    W1c = jnp.pad(W1[:, :_F * _D].reshape(_H, _F, _D),
                  ((0, 0), (0, 0), (0, _DP - _D))).reshape(_H, _KC)
    W1n = W1[:, _F * _D:]
    return _mlp(x_c, x_num, W_num, b_num.reshape(1, _NUMD), W1c, W1n,
                b1.reshape(1, _H), W2, b2.reshape(1, 3))


# speedup vs baseline: 2.5884x; 1.0527x over previous
import functools

import jax
import jax.numpy as jnp
from jax import lax
from jax.experimental import pallas as pl
from jax.experimental.pallas import tpu as pltpu
from jax.experimental.pallas import tpu_sc as plsc

_B = 16384
_F = 26
_V = 100000
_D = 50
_DP = 64
_NUMF = 13
_NUMD = 130
_H = 715

_N = _B * _F
_NC = 2
_NS = 16
_NW = _NC * _NS
_PER_W = _N // _NW
_LPC = 128
_CH = _PER_W // _LPC
_VC = 10240
_NB = -(-_V // _VC)
_RPB = _VC // 2
_RPF = _NB * _RPB
_TR = _F * _RPF
_CPW = _LPC * _DP
_CPR = _CPW // 128
_OUTR = _N * _DP // 128


def _repack_body(tin_ref, out_ref):
    x = tin_ref[0]
    xt = jnp.swapaxes(x, 0, 1)
    out_ref[:, 0:_D] = xt[0:_RPB]
    out_ref[:, _D:_DP] = jnp.zeros((_RPB, _DP - _D), jnp.float32)
    out_ref[:, _DP:_DP + _D] = xt[_RPB:_VC]
    out_ref[:, _DP + _D:128] = jnp.zeros((_RPB, _DP - _D), jnp.float32)


def _repack(tablesT):
    return pl.pallas_call(
        _repack_body,
        grid=(_F, _NB),
        in_specs=[pl.BlockSpec((1, _D, _VC), lambda f, v: (f, 0, v))],
        out_specs=pl.BlockSpec((_RPB, 128), lambda f, v: (f * _NB + v, 0)),
        out_shape=jax.ShapeDtypeStruct((_TR, 128), jnp.float32),
        compiler_params=pltpu.CompilerParams(
            dimension_semantics=("arbitrary", "arbitrary"),
        ),
    )(tablesT)


def _copy_chunk(buf, off_v, cp, jj, iota):
    @pl.loop(0, _LPC // 16)
    def _(g):
        off16 = off_v[jj, pl.ds(g * 16, 16)]
        for c in range(16):
            off = pl.multiple_of(off16[c], 16)
            i = g * 16 + c
            for k in range(4):
                v = buf[i, pl.ds(off + 16 * k, 16)]
                w = c * _DP + k * 16
                cp[g * 8 + w // 128, pl.ds(w % 128, 16)] = v


def _gather_body(table_hbm, rows_hbm, off_hbm, out_hbm,
                 rows_v, off_v, bufs, cps, gsem, osem):
    c = lax.axis_index("c")
    s = lax.axis_index("s")
    wid = s * _NC + c
    wbase = wid * (_CH * _CPR)
    iota = lax.iota(jnp.int32, 16)

    pltpu.sync_copy(rows_hbm.at[wid], rows_v)
    pltpu.sync_copy(off_hbm.at[wid], off_v)

    def gather(jj, p):
        pltpu.async_copy(table_hbm.at[rows_v.at[jj]], bufs[p], gsem[p])

    def gather_wait(jj, p):
        pltpu.make_async_copy(table_hbm.at[rows_v.at[jj]], bufs[p],
                              gsem[p]).wait()

    def out_desc(jj, p):
        dst = pl.multiple_of(wbase + jj * _CPR, 8)
        return pltpu.make_async_copy(cps[p], out_hbm.at[pl.ds(dst, _CPR)],
                                     osem[p])

    gather(0, 0)

    @pl.loop(0, _CH, step=2)
    def _(j):
        for ci in range(2):
            jj = j + ci
            p = ci
            gather_wait(jj, p)

            @pl.when(jj + 1 < _CH)
            def _():
                gather(jj + 1, 1 - p)

            @pl.when(jj >= 2)
            def _():
                out_desc(jj - 2, p).wait()

            _copy_chunk(bufs[p], off_v, cps[p], jj, iota)
            out_desc(jj, p).start()

    out_desc(_CH - 2, 0).wait()
    out_desc(_CH - 1, 1).wait()


_sc_gather = functools.partial(
    pl.kernel,
    out_type=jax.ShapeDtypeStruct((_OUTR, 128), jnp.float32),
    mesh=plsc.VectorSubcoreMesh(core_axis_name="c", subcore_axis_name="s"),
    scratch_types=[
        pltpu.VMEM((_CH, _LPC), jnp.int32),
        pltpu.VMEM((_CH, _LPC), jnp.int32),
        [pltpu.VMEM((_LPC, 128), jnp.float32),
         pltpu.VMEM((_LPC, 128), jnp.float32)],
        [pltpu.VMEM((_CPR, 128), jnp.float32),
         pltpu.VMEM((_CPR, 128), jnp.float32)],
        [pltpu.SemaphoreType.DMA, pltpu.SemaphoreType.DMA],
        [pltpu.SemaphoreType.DMA, pltpu.SemaphoreType.DMA],
    ],
)(_gather_body)


_BM = 1024
_KC = _F * _DP


def _mlp_body(xc_ref, xn_ref, wnum_ref, bnum_ref, w1c_ref, w1n_ref, b1_ref,
              w2_ref, b2_ref, out_ref):
    x_n = lax.dot_general(xn_ref[...], wnum_ref[...],
                          (((1,), (1,)), ((), ()))) + bnum_ref[...]
    hc = lax.dot_general(xc_ref[...], w1c_ref[...], (((1,), (1,)), ((), ())))
    hn = lax.dot_general(x_n, w1n_ref[...], (((1,), (1,)), ((), ())))
    h = jnp.maximum(hc + hn + b1_ref[...], 0.0)
    out_ref[...] = lax.dot_general(h, w2_ref[...],
                                   (((1,), (1,)), ((), ()))) + b2_ref[...]


def _mlp(x_c, x_num, W_num, b_num, W1c, W1n, b1, W2, b2):
    return pl.pallas_call(
        _mlp_body,
        grid=(_B // _BM,),
        in_specs=[
            pl.BlockSpec((_BM, _KC), lambda i: (i, 0)),
            pl.BlockSpec((_BM, _NUMF), lambda i: (i, 0)),
            pl.BlockSpec((_NUMD, _NUMF), lambda i: (0, 0)),
            pl.BlockSpec((1, _NUMD), lambda i: (0, 0)),
            pl.BlockSpec((_H, _KC), lambda i: (0, 0)),
            pl.BlockSpec((_H, _NUMD), lambda i: (0, 0)),
            pl.BlockSpec((1, _H), lambda i: (0, 0)),
            pl.BlockSpec((3, _H), lambda i: (0, 0)),
            pl.BlockSpec((1, 3), lambda i: (0, 0)),
        ],
        out_specs=pl.BlockSpec((_BM, 3), lambda i: (i, 0)),
        out_shape=jax.ShapeDtypeStruct((_B, 3), jnp.float32),
        compiler_params=pltpu.CompilerParams(
            dimension_semantics=("arbitrary",),
        ),
    )(x_c, x_num, W_num, b_num, W1c, W1n, b1, W2, b2)


def kernel(x_cat, x_num, tables, W_num, b_num, W1, b1, W2, b2):
    tablesT = jnp.swapaxes(tables, 1, 2)
    table128 = _repack(tablesT)
    v = x_cat.astype(jnp.int32)
    f = jnp.arange(_F, dtype=jnp.int32)[None, :]
    row = (f * _RPF + (v // _VC) * _RPB + (v % _RPB)).reshape(-1)
    off = (((v % _VC) // _RPB) * _DP).reshape(-1)
    rows = row.reshape(_NW, _CH, _LPC)
    offs = off.reshape(_NW, _CH, _LPC)
    flat = _sc_gather(table128, rows, offs)
    x_c = flat.reshape(_B, _KC)
    W1c = jnp.pad(W1[:, :_F * _D].reshape(_H, _F, _D),
                  ((0, 0), (0, 0), (0, _DP - _D))).reshape(_H, _KC)
    W1n = W1[:, _F * _D:]
    return _mlp(x_c, x_num, W_num, b_num.reshape(1, _NUMD), W1c, W1n,
                b1.reshape(1, _H), W2, b2.reshape(1, 3))


# speedup vs baseline: 2.6596x; 1.0275x over previous
import functools

import jax
import jax.numpy as jnp
from jax import lax
from jax.experimental import pallas as pl
from jax.experimental.pallas import tpu as pltpu
from jax.experimental.pallas import tpu_sc as plsc

_B = 16384
_F = 26
_V = 100000
_D = 50
_DP = 64
_NUMF = 13
_NUMD = 130
_H = 715

_N = _B * _F
_NC = 2
_NS = 16
_NW = _NC * _NS
_PER_W = _N // _NW
_LPC = 128
_CH = _PER_W // _LPC
_VC = 12800
_NB = -(-_V // _VC)
_RPB = _VC // 2
_RPF = _NB * _RPB
_TR = _F * _RPF
_CPW = _LPC * _DP
_CPR = _CPW // 128
_OUTR = _N * _DP // 128


def _repack_body(tin_ref, out_ref):
    x = tin_ref[0]
    xt = jnp.swapaxes(x, 0, 1)
    out_ref[:, 0:_D] = xt[0:_RPB]
    out_ref[:, _D:_DP] = jnp.zeros((_RPB, _DP - _D), jnp.float32)
    out_ref[:, _DP:_DP + _D] = xt[_RPB:_VC]
    out_ref[:, _DP + _D:128] = jnp.zeros((_RPB, _DP - _D), jnp.float32)


def _repack(tablesT):
    return pl.pallas_call(
        _repack_body,
        grid=(_F, _NB),
        in_specs=[pl.BlockSpec((1, _D, _VC), lambda f, v: (f, 0, v))],
        out_specs=pl.BlockSpec((_RPB, 128), lambda f, v: (f * _NB + v, 0)),
        out_shape=jax.ShapeDtypeStruct((_TR, 128), jnp.float32),
        compiler_params=pltpu.CompilerParams(
            dimension_semantics=("arbitrary", "arbitrary"),
        ),
    )(tablesT)


def _copy_chunk(buf, off_v, cp, jj, iota):
    @pl.loop(0, _LPC // 16)
    def _(g):
        off16 = off_v[jj, pl.ds(g * 16, 16)]
        for c in range(16):
            off = pl.multiple_of(off16[c], 16)
            i = g * 16 + c
            for k in range(4):
                v = buf[i, pl.ds(off + 16 * k, 16)]
                w = c * _DP + k * 16
                cp[g * 8 + w // 128, pl.ds(w % 128, 16)] = v


def _gather_body(table_hbm, rows_hbm, off_hbm, out_hbm,
                 rows_v, off_v, bufs, cps, gsem, osem):
    c = lax.axis_index("c")
    s = lax.axis_index("s")
    wid = s * _NC + c
    wbase = wid * (_CH * _CPR)
    iota = lax.iota(jnp.int32, 16)

    pltpu.sync_copy(rows_hbm.at[wid], rows_v)
    pltpu.sync_copy(off_hbm.at[wid], off_v)

    def gather(jj, p):
        pltpu.async_copy(table_hbm.at[rows_v.at[jj]], bufs[p], gsem[p])

    def gather_wait(jj, p):
        pltpu.make_async_copy(table_hbm.at[rows_v.at[jj]], bufs[p],
                              gsem[p]).wait()

    def out_desc(jj, p):
        dst = pl.multiple_of(wbase + jj * _CPR, 8)
        return pltpu.make_async_copy(cps[p], out_hbm.at[pl.ds(dst, _CPR)],
                                     osem[p])

    gather(0, 0)

    @pl.loop(0, _CH, step=2)
    def _(j):
        for ci in range(2):
            jj = j + ci
            p = ci
            gather_wait(jj, p)

            @pl.when(jj + 1 < _CH)
            def _():
                gather(jj + 1, 1 - p)

            @pl.when(jj >= 2)
            def _():
                out_desc(jj - 2, p).wait()

            _copy_chunk(bufs[p], off_v, cps[p], jj, iota)
            out_desc(jj, p).start()

    out_desc(_CH - 2, 0).wait()
    out_desc(_CH - 1, 1).wait()


_sc_gather = functools.partial(
    pl.kernel,
    out_type=jax.ShapeDtypeStruct((_OUTR, 128), jnp.float32),
    mesh=plsc.VectorSubcoreMesh(core_axis_name="c", subcore_axis_name="s"),
    scratch_types=[
        pltpu.VMEM((_CH, _LPC), jnp.int32),
        pltpu.VMEM((_CH, _LPC), jnp.int32),
        [pltpu.VMEM((_LPC, 128), jnp.float32),
         pltpu.VMEM((_LPC, 128), jnp.float32)],
        [pltpu.VMEM((_CPR, 128), jnp.float32),
         pltpu.VMEM((_CPR, 128), jnp.float32)],
        [pltpu.SemaphoreType.DMA, pltpu.SemaphoreType.DMA],
        [pltpu.SemaphoreType.DMA, pltpu.SemaphoreType.DMA],
    ],
)(_gather_body)


_BM = 1024
_KC = _F * _DP


def _mlp_body(xc_ref, xn_ref, wnum_ref, bnum_ref, w1c_ref, w1n_ref, b1_ref,
              w2_ref, b2_ref, out_ref):
    x_n = lax.dot_general(xn_ref[...], wnum_ref[...],
                          (((1,), (1,)), ((), ()))) + bnum_ref[...]
    hc = lax.dot_general(xc_ref[...], w1c_ref[...], (((1,), (1,)), ((), ())))
    hn = lax.dot_general(x_n, w1n_ref[...], (((1,), (1,)), ((), ())))
    h = jnp.maximum(hc + hn + b1_ref[...], 0.0)
    out_ref[...] = lax.dot_general(h, w2_ref[...],
                                   (((1,), (1,)), ((), ()))) + b2_ref[...]


def _mlp(x_c, x_num, W_num, b_num, W1c, W1n, b1, W2, b2):
    return pl.pallas_call(
        _mlp_body,
        grid=(_B // _BM,),
        in_specs=[
            pl.BlockSpec((_BM, _KC), lambda i: (i, 0)),
            pl.BlockSpec((_BM, _NUMF), lambda i: (i, 0)),
            pl.BlockSpec((_NUMD, _NUMF), lambda i: (0, 0)),
            pl.BlockSpec((1, _NUMD), lambda i: (0, 0)),
            pl.BlockSpec((_H, _KC), lambda i: (0, 0)),
            pl.BlockSpec((_H, _NUMD), lambda i: (0, 0)),
            pl.BlockSpec((1, _H), lambda i: (0, 0)),
            pl.BlockSpec((3, _H), lambda i: (0, 0)),
            pl.BlockSpec((1, 3), lambda i: (0, 0)),
        ],
        out_specs=pl.BlockSpec((_BM, 3), lambda i: (i, 0)),
        out_shape=jax.ShapeDtypeStruct((_B, 3), jnp.float32),
        compiler_params=pltpu.CompilerParams(
            dimension_semantics=("arbitrary",),
        ),
    )(x_c, x_num, W_num, b_num, W1c, W1n, b1, W2, b2)


def kernel(x_cat, x_num, tables, W_num, b_num, W1, b1, W2, b2):
    tablesT = jnp.swapaxes(tables, 1, 2)
    table128 = _repack(tablesT)
    v = x_cat.astype(jnp.int32)
    f = jnp.arange(_F, dtype=jnp.int32)[None, :]
    row = (f * _RPF + (v // _VC) * _RPB + (v % _RPB)).reshape(-1)
    off = (((v % _VC) // _RPB) * _DP).reshape(-1)
    rows = row.reshape(_NW, _CH, _LPC)
    offs = off.reshape(_NW, _CH, _LPC)
    flat = _sc_gather(table128, rows, offs)
    x_c = flat.reshape(_B, _KC)
    W1c = jnp.pad(W1[:, :_F * _D].reshape(_H, _F, _D),
                  ((0, 0), (0, 0), (0, _DP - _D))).reshape(_H, _KC)
    W1n = W1[:, _F * _D:]
    return _mlp(x_c, x_num, W_num, b_num.reshape(1, _NUMD), W1c, W1n,
                b1.reshape(1, _H), W2, b2.reshape(1, 3))
